# initial kernel scaffold (unmeasured)
import jax
import jax.numpy as jnp
from jax import lax
from jax.experimental import pallas as pl
from jax.experimental.pallas import tpu as pltpu

N_DEV = 8
N_CW = 4
N_CCW = 3


def kernel(x, w_mat, scale_x, scale_w):
    m_per, k = x.shape
    _, n_per = w_mat.shape

    x8 = x.astype(jnp.float8_e4m3fn)
    w8 = w_mat.astype(jnp.float8_e4m3fn)
    s = (scale_x[0] * scale_w[0]).reshape(1, 1)

    def body(x_ref, w_ref, s_ref, out_ref, gx_ref,
             cw_send, cw_recv, ccw_send, ccw_recv):
        my = lax.axis_index("i")

        def mod(v):
            return lax.rem(v + N_DEV, N_DEV)

        left = mod(my - 1)
        right = mod(my + 1)

        barrier = pltpu.get_barrier_semaphore()
        for nbr in (left, right):
            pl.semaphore_signal(barrier, inc=1, device_id=(nbr,),
                                device_id_type=pl.DeviceIdType.MESH)
        pl.semaphore_wait(barrier, 2)

        scale = s_ref[0, 0]

        def rows(o):
            return pl.ds(o * m_per, m_per)

        def gemm(o):
            a = gx_ref[rows(o), :]
            acc = jnp.dot(a, w_ref[:, :], preferred_element_type=jnp.float32)
            y = acc * scale
            out_ref[rows(o), :] = y * jax.nn.sigmoid(y)

        def mk(o, dest, ssem, rsem):
            return pltpu.make_async_remote_copy(
                src_ref=gx_ref.at[rows(o)],
                dst_ref=gx_ref.at[rows(o)],
                send_sem=ssem,
                recv_sem=rsem,
                device_id=(dest,),
                device_id_type=pl.DeviceIdType.MESH,
            )

        gx_ref[rows(my), :] = x_ref[:, :]

        cw = mk(my, right, cw_send.at[0], cw_recv.at[0])
        cw.start()
        ccw = mk(my, left, ccw_send.at[0], ccw_recv.at[0])
        ccw.start()
        gemm(my)

        for h in range(1, N_CW):
            cw.wait()
            cw = mk(mod(my - h), right, cw_send.at[h], cw_recv.at[h])
            cw.start()
            if h < N_CCW:
                ccw.wait()
                ccw = mk(mod(my + h), left, ccw_send.at[h], ccw_recv.at[h])
                ccw.start()
            gemm(mod(my - h))
            if h < N_CCW:
                gemm(mod(my + h))

        ccw.wait()
        gemm(mod(my + N_CCW))
        cw.wait()
        gemm(mod(my - N_CW))

    out_shape = jax.ShapeDtypeStruct((N_DEV * m_per, n_per), jnp.float32)
    return pl.pallas_call(
        body,
        out_shape=out_shape,
        in_specs=[
            pl.BlockSpec(memory_space=pltpu.VMEM),
            pl.BlockSpec(memory_space=pltpu.VMEM),
            pl.BlockSpec(memory_space=pltpu.SMEM),
        ],
        out_specs=pl.BlockSpec(memory_space=pltpu.VMEM),
        scratch_shapes=[
            pltpu.VMEM((N_DEV * m_per, k), jnp.float8_e4m3fn),
            pltpu.SemaphoreType.DMA((N_CW,)),
            pltpu.SemaphoreType.DMA((N_CW,)),
            pltpu.SemaphoreType.DMA((N_CCW,)),
            pltpu.SemaphoreType.DMA((N_CCW,)),
        ],
        compiler_params=pltpu.CompilerParams(collective_id=0),
    )(x8, w8, s)


# baseline (device time: 140128 ns/iter reference)
import jax
import jax.numpy as jnp
from jax import lax
from jax.experimental import pallas as pl
from jax.experimental.pallas import tpu as pltpu

N_DEV = 8
N_CW = 4
N_CCW = 3


def kernel(x, w_mat, scale_x, scale_w):
    m_per, k = x.shape
    _, n_per = w_mat.shape

    x8 = x.astype(jnp.float8_e4m3fn)
    w8 = w_mat.astype(jnp.float8_e4m3fn)
    s = (scale_x[0] * scale_w[0]).reshape(1, 1)

    def body(x_ref, w_ref, s_ref, out_ref, gx_ref,
             cw_send, cw_recv, ccw_send, ccw_recv):
        my = lax.axis_index("i")

        def mod(v):
            return lax.rem(v + N_DEV, N_DEV)

        left = mod(my - 1)
        right = mod(my + 1)

        barrier = pltpu.get_barrier_semaphore()
        for nbr in (left, right):
            pl.semaphore_signal(barrier, inc=1, device_id=(nbr,),
                                device_id_type=pl.DeviceIdType.MESH)
        pl.semaphore_wait(barrier, 2)

        scale = s_ref[0, 0]

        def rows(o):
            return pl.ds(o * m_per, m_per)

        def gemm(o):
            a = gx_ref[rows(o), :]
            acc = jnp.dot(a, w_ref[:, :], preferred_element_type=jnp.float32)
            y = acc * scale
            out_ref[rows(o), :] = y * jax.nn.sigmoid(y)

        def mk(o, dest, ssem, rsem):
            return pltpu.make_async_remote_copy(
                src_ref=gx_ref.at[rows(o)],
                dst_ref=gx_ref.at[rows(o)],
                send_sem=ssem,
                recv_sem=rsem,
                device_id=(dest,),
                device_id_type=pl.DeviceIdType.MESH,
            )

        gx_ref[rows(my), :] = x_ref[:, :]

        cw = mk(my, right, cw_send.at[0], cw_recv.at[0])
        cw.start()
        ccw = mk(my, left, ccw_send.at[0], ccw_recv.at[0])
        ccw.start()
        gemm(my)

        for h in range(1, N_CW):
            cw.wait()
            cw = mk(mod(my - h), right, cw_send.at[h], cw_recv.at[h])
            cw.start()
            if h < N_CCW:
                ccw.wait()
                ccw = mk(mod(my + h), left, ccw_send.at[h], ccw_recv.at[h])
                ccw.start()
            gemm(mod(my - h))
            if h < N_CCW:
                gemm(mod(my + h))

        ccw.wait()
        gemm(mod(my + N_CCW))
        cw.wait()
        gemm(mod(my - N_CW))

    out_shape = jax.ShapeDtypeStruct((N_DEV * m_per, n_per), jnp.float32)
    return pl.pallas_call(
        body,
        out_shape=out_shape,
        in_specs=[
            pl.BlockSpec(memory_space=pltpu.VMEM),
            pl.BlockSpec(memory_space=pltpu.VMEM),
            pl.BlockSpec(memory_space=pltpu.SMEM),
        ],
        out_specs=pl.BlockSpec(memory_space=pltpu.VMEM),
        scratch_shapes=[
            pltpu.VMEM((N_DEV * m_per, k), jnp.float8_e4m3fn),
            pltpu.SemaphoreType.DMA((N_CW,)),
            pltpu.SemaphoreType.DMA((N_CW,)),
            pltpu.SemaphoreType.DMA((N_CCW,)),
            pltpu.SemaphoreType.DMA((N_CCW,)),
        ],
        compiler_params=pltpu.CompilerParams(
            collective_id=0,
            vmem_limit_bytes=100 * 1024 * 1024,
        ),
    )(x8, w8, s)


# device time: 109938 ns/iter; 1.2746x vs baseline; 1.2746x over previous
import jax
import jax.numpy as jnp
from jax import lax
from jax.experimental import pallas as pl
from jax.experimental.pallas import tpu as pltpu

N_DEV = 8
PLANE = 4


def kernel(x, w_mat, scale_x, scale_w):
    m_per, k = x.shape
    _, n_per = w_mat.shape
    half = m_per // 2
    s = jnp.reshape(scale_x * scale_w, (1, 1))

    def body(x_ref, w_ref, s_ref, out_ref, gx_ref, w8_ref,
             cw_send, cw_recv, ccw_send, ccw_recv, z_send, z_recv):
        my = lax.axis_index("i")
        z = lax.div(my, PLANE)
        p = lax.rem(my, PLANE)

        def ring(q):
            return z * PLANE + lax.rem(q + PLANE, PLANE)

        def xring(q):
            return (1 - z) * PLANE + lax.rem(q + PLANE, PLANE)

        left = ring(p - 1)
        right = ring(p + 1)
        corner = ring(p + 2)
        partner = xring(p)
        l_partner = xring(p - 1)
        r_partner = xring(p + 1)
        x_corner = xring(p + 2)

        barrier = pltpu.get_barrier_semaphore()
        for nbr in (left, right, partner):
            pl.semaphore_signal(barrier, inc=1, device_id=(nbr,),
                                device_id_type=pl.DeviceIdType.MESH)
        pl.semaphore_wait(barrier, 3)

        scale = s_ref[0, 0]

        def rows(o):
            return pl.ds(o * m_per, m_per)

        def rows_half(o, h):
            return pl.ds(o * m_per + h * half, half)

        def gemm(o):
            a = gx_ref[rows(o), :]
            acc = jnp.dot(a, w8_ref[:, :], preferred_element_type=jnp.float32)
            y = acc * scale
            out_ref[rows(o), :] = y * jax.nn.sigmoid(y)

        def mk(sl, dest, ssem, rsem):
            return pltpu.make_async_remote_copy(
                src_ref=gx_ref.at[sl],
                dst_ref=gx_ref.at[sl],
                send_sem=ssem,
                recv_sem=rsem,
                device_id=(dest,),
                device_id_type=pl.DeviceIdType.MESH,
            )

        gx_ref[rows(my), :] = x_ref[:, :].astype(jnp.float8_e4m3fn)

        cw0 = mk(rows(my), right, cw_send.at[0], cw_recv.at[0])
        cw0.start()
        ccw0 = mk(rows(my), left, ccw_send.at[0], ccw_recv.at[0])
        ccw0.start()
        z0 = mk(rows(my), partner, z_send.at[0], z_recv.at[0])
        z0.start()

        w8_ref[:, :] = w_ref[:, :].astype(jnp.float8_e4m3fn)
        gemm(my)

        cw0.wait()
        cw1 = mk(rows_half(left, 0), right, cw_send.at[1], cw_recv.at[1])
        cw1.start()
        ccw0.wait()
        ccw1 = mk(rows_half(right, 1), left, ccw_send.at[1], ccw_recv.at[1])
        ccw1.start()
        z0.wait()
        cw2 = mk(rows(partner), right, cw_send.at[2], cw_recv.at[2])
        cw2.start()
        ccw2 = mk(rows(partner), left, ccw_send.at[2], ccw_recv.at[2])
        ccw2.start()

        gemm(left)
        gemm(right)
        gemm(partner)

        cw1.wait()
        ccw1.wait()
        gemm(corner)

        cw2.wait()
        cw3 = mk(rows_half(l_partner, 0), right, cw_send.at[3], cw_recv.at[3])
        cw3.start()
        gemm(l_partner)
        ccw2.wait()
        ccw3 = mk(rows_half(r_partner, 1), left, ccw_send.at[3], ccw_recv.at[3])
        ccw3.start()
        gemm(r_partner)

        cw3.wait()
        ccw3.wait()
        gemm(x_corner)

    out_shape = jax.ShapeDtypeStruct((N_DEV * m_per, n_per), jnp.float32)
    return pl.pallas_call(
        body,
        out_shape=out_shape,
        in_specs=[
            pl.BlockSpec(memory_space=pltpu.VMEM),
            pl.BlockSpec(memory_space=pltpu.VMEM),
            pl.BlockSpec(memory_space=pltpu.SMEM),
        ],
        out_specs=pl.BlockSpec(memory_space=pltpu.VMEM),
        scratch_shapes=[
            pltpu.VMEM((N_DEV * m_per, k), jnp.float8_e4m3fn),
            pltpu.VMEM((k, n_per), jnp.float8_e4m3fn),
            pltpu.SemaphoreType.DMA((4,)),
            pltpu.SemaphoreType.DMA((4,)),
            pltpu.SemaphoreType.DMA((4,)),
            pltpu.SemaphoreType.DMA((4,)),
            pltpu.SemaphoreType.DMA((1,)),
            pltpu.SemaphoreType.DMA((1,)),
        ],
        compiler_params=pltpu.CompilerParams(
            collective_id=0,
            vmem_limit_bytes=100 * 1024 * 1024,
        ),
    )(x, w_mat, s)


# device time: 96770 ns/iter; 1.4481x vs baseline; 1.1361x over previous
import jax
import jax.numpy as jnp
from jax import lax
from jax.experimental import pallas as pl
from jax.experimental.pallas import tpu as pltpu

N_DEV = 8
PLANE = 4
SLIVER = 64


def kernel(x, w_mat, scale_x, scale_w):
    m_per, k = x.shape
    _, n_per = w_mat.shape
    half = m_per // 2
    big = m_per - SLIVER
    s = jnp.reshape(scale_x * scale_w, (1, 1))

    def body(x_ref, w_ref, s_ref, out_ref, gx_ref, w8_ref,
             cw_send, cw_recv, ccw_send, ccw_recv, z_send, z_recv):
        my = lax.axis_index("i")
        zz = lax.div(my, PLANE)
        p = lax.rem(my, PLANE)

        def ring(q):
            return zz * PLANE + lax.rem(q + PLANE, PLANE)

        def xring(q):
            return (1 - zz) * PLANE + lax.rem(q + PLANE, PLANE)

        left = ring(p - 1)
        right = ring(p + 1)
        corner = ring(p + 2)
        partner = xring(p)
        l_partner = xring(p - 1)
        r_partner = xring(p + 1)
        x_corner = xring(p + 2)

        barrier = pltpu.get_barrier_semaphore()
        for nbr in (left, right, partner):
            pl.semaphore_signal(barrier, inc=1, device_id=(nbr,),
                                device_id_type=pl.DeviceIdType.MESH)
        pl.semaphore_wait(barrier, 3)

        scale = s_ref[0, 0]

        def rows(o, lo=0, num=None):
            return pl.ds(o * m_per + lo, m_per if num is None else num)

        def gemm(o):
            a = gx_ref[rows(o), :]
            acc = jnp.dot(a, w8_ref[:, :], preferred_element_type=jnp.float32)
            y = acc * scale
            out_ref[rows(o), :] = y * jax.nn.sigmoid(y)

        def mk(sl, dest, ssem, rsem):
            return pltpu.make_async_remote_copy(
                src_ref=gx_ref.at[sl],
                dst_ref=gx_ref.at[sl],
                send_sem=ssem,
                recv_sem=rsem,
                device_id=(dest,),
                device_id_type=pl.DeviceIdType.MESH,
            )

        gx_ref[rows(my), :] = x_ref[:, :].astype(jnp.float8_e4m3fn)

        cw0 = mk(rows(my), right, cw_send.at[0], cw_recv.at[0])
        cw0.start()
        ccw0 = mk(rows(my), left, ccw_send.at[0], ccw_recv.at[0])
        ccw0.start()
        z0 = mk(rows(my), partner, z_send.at[0], z_recv.at[0])
        z0.start()

        w8_ref[:, :] = w_ref[:, :].astype(jnp.float8_e4m3fn)
        gemm(my)

        cw0.wait_recv()
        cw1 = mk(rows(left, 0, half), right, cw_send.at[1], cw_recv.at[1])
        cw1.start()
        z1 = mk(rows(left, big, SLIVER), partner, z_send.at[1], z_recv.at[1])
        z1.start()

        ccw0.wait_recv()
        ccw1 = mk(rows(right, half, half), left, ccw_send.at[1], ccw_recv.at[1])
        ccw1.start()
        z2 = mk(rows(right, 0, SLIVER), partner, z_send.at[2], z_recv.at[2])
        z2.start()

        z0.wait_recv()
        cw2 = mk(rows(partner, 0, big), right, cw_send.at[2], cw_recv.at[2])
        cw2.start()
        ccw2 = mk(rows(partner, SLIVER, big), left, ccw_send.at[2], ccw_recv.at[2])
        ccw2.start()

        gemm(left)
        gemm(right)
        gemm(partner)

        cw1.wait_recv()
        ccw1.wait_recv()
        z3 = mk(rows(corner), partner, z_send.at[3], z_recv.at[3])
        z3.start()
        gemm(corner)

        cw2.wait_recv()
        z1.wait_recv()
        gemm(l_partner)

        ccw2.wait_recv()
        z2.wait_recv()
        gemm(r_partner)

        z3.wait_recv()
        gemm(x_corner)

        for r in (cw0, cw1, cw2, ccw0, ccw1, ccw2, z0, z1, z2, z3):
            r.wait_send()

    out_shape = jax.ShapeDtypeStruct((N_DEV * m_per, n_per), jnp.float32)
    return pl.pallas_call(
        body,
        out_shape=out_shape,
        in_specs=[
            pl.BlockSpec(memory_space=pltpu.VMEM),
            pl.BlockSpec(memory_space=pltpu.VMEM),
            pl.BlockSpec(memory_space=pltpu.SMEM),
        ],
        out_specs=pl.BlockSpec(memory_space=pltpu.VMEM),
        scratch_shapes=[
            pltpu.VMEM((N_DEV * m_per, k), jnp.float8_e4m3fn),
            pltpu.VMEM((k, n_per), jnp.float8_e4m3fn),
            pltpu.SemaphoreType.DMA((3,)),
            pltpu.SemaphoreType.DMA((3,)),
            pltpu.SemaphoreType.DMA((3,)),
            pltpu.SemaphoreType.DMA((3,)),
            pltpu.SemaphoreType.DMA((4,)),
            pltpu.SemaphoreType.DMA((4,)),
        ],
        compiler_params=pltpu.CompilerParams(
            collective_id=0,
            vmem_limit_bytes=100 * 1024 * 1024,
        ),
    )(x, w_mat, s)
